# baseline (speedup 1.0000x reference)
import functools

import jax
import jax.numpy as jnp
from jax import lax
from jax.experimental import pallas as pl
from jax.experimental.pallas import tpu as pltpu
from jax.experimental.pallas import tpu_sc as plsc

D = 32
L = 16
NW = 32
CH = 128
EPS = 1e-5


def _rsqrt(x):
    i = plsc.bitcast(x, jnp.int32)
    i = jnp.int32(0x5F3759DF) - lax.shift_right_logical(i, 1)
    y = plsc.bitcast(i, jnp.float32)
    for _ in range(3):
        y = y * (1.5 - 0.5 * x * y * y)
    return y


def _sc_body(per_w, table_hbm, idx_hbm, g_hbm, b_hbm, out_hbm,
             idx_v, rows_v, g_v, b_v, sem):
    wid = lax.axis_index("s") * 2 + lax.axis_index("c")
    slab = wid * per_w

    pltpu.sync_copy(idx_hbm.at[pl.ds(slab, per_w)], idx_v)
    pltpu.sync_copy(g_hbm, g_v)
    pltpu.sync_copy(b_hbm, b_v)

    n_chunks = per_w // CH
    lane_iota = lax.iota(jnp.int32, L)

    def group_body(grp, carry):
        row_ids = lane_iota + grp * L
        s = jnp.zeros((L,), jnp.float32)
        sq = jnp.zeros((L,), jnp.float32)
        for j in range(D):
            col = jnp.full((L,), j, jnp.int32)
            v = plsc.load_gather(rows_v, [row_ids, col])
            s = s + v
            sq = sq + v * v
        mean = s * (1.0 / D)
        var = sq * (1.0 / D) - mean * mean
        rstd = _rsqrt(var + EPS)
        for j in range(D):
            col = jnp.full((L,), j, jnp.int32)
            v = plsc.load_gather(rows_v, [row_ids, col])
            o = (v - mean) * rstd * g_v[j, :] + b_v[j, :]
            plsc.store_scatter(rows_v, [row_ids, col], o)
        return carry

    def chunk_body(c, carry):
        base = c * CH
        pltpu.async_copy(table_hbm.at[idx_v.at[pl.ds(base, CH)]],
                         rows_v, sem).wait()
        lax.fori_loop(0, CH // L, group_body, 0)
        pltpu.sync_copy(rows_v, out_hbm.at[pl.ds(slab + base, CH)])
        return carry

    lax.fori_loop(0, n_chunks, chunk_body, 0)


def kernel(ticker_ids, table, gamma, beta):
    if ticker_ids.ndim == 2 and ticker_ids.shape[-1] == 1:
        ticker_ids = ticker_ids[..., 0]
    out_shape = ticker_ids.shape + (D,)
    flat = ticker_ids.reshape(-1).astype(jnp.int32)
    n = flat.shape[0]

    unit = NW * CH
    n_pad = ((n + unit - 1) // unit) * unit
    if n_pad != n:
        flat = jnp.concatenate(
            [flat, jnp.zeros((n_pad - n,), jnp.int32)])
    per_w = n_pad // NW

    g_t = jnp.broadcast_to(gamma.astype(jnp.float32)[:, None], (D, L))
    b_t = jnp.broadcast_to(beta.astype(jnp.float32)[:, None], (D, L))

    mesh = plsc.VectorSubcoreMesh(core_axis_name="c", subcore_axis_name="s",
                                  num_cores=2, num_subcores=16)
    run = pl.kernel(
        functools.partial(_sc_body, per_w),
        out_type=jax.ShapeDtypeStruct((n_pad, D), jnp.float32),
        mesh=mesh,
        compiler_params=pltpu.CompilerParams(needs_layout_passes=False,
                                             use_tc_tiling_on_sc=False),
        scratch_types=[
            pltpu.VMEM((per_w,), jnp.int32),
            pltpu.VMEM((CH, D), jnp.float32),
            pltpu.VMEM((D, L), jnp.float32),
            pltpu.VMEM((D, L), jnp.float32),
            pltpu.SemaphoreType.DMA,
        ],
    )
    out = run(table.astype(jnp.float32), flat, g_t, b_t)
    return out[:n].reshape(out_shape)


# speedup vs baseline: 1.0011x; 1.0011x over previous
import functools

import jax
import jax.numpy as jnp
from jax import lax
from jax.experimental import pallas as pl
from jax.experimental.pallas import tpu as pltpu
from jax.experimental.pallas import tpu_sc as plsc

D = 32
L = 16
NW = 32
CH = 128
NG = CH // L
EPS = 1e-5


def _rsqrt(x):
    i = plsc.bitcast(x, jnp.int32)
    i = jnp.int32(0x5F3759DF) - lax.shift_right_logical(i, 1)
    y = plsc.bitcast(i, jnp.float32)
    for _ in range(3):
        y = y * (1.5 - 0.5 * x * y * y)
    return y


def _sc_body(per_w, table_hbm, idx_hbm, g_hbm, b_hbm, out_hbm,
             idx_v, rows0, rows1, g_v, b_v, gsem0, gsem1, osem0, osem1):
    wid = lax.axis_index("s") * 2 + lax.axis_index("c")
    slab = wid * per_w

    pltpu.sync_copy(idx_hbm.at[pl.ds(slab, per_w)], idx_v)
    pltpu.sync_copy(g_hbm, g_v)
    pltpu.sync_copy(b_hbm, b_v)

    n_chunks = per_w // CH
    lane_iota = lax.iota(jnp.int32, L)
    row_ids = [lane_iota + g * L for g in range(NG)]

    def gather(c, rows_v, sem):
        pltpu.make_async_copy(
            table_hbm.at[idx_v.at[pl.ds(c * CH, CH)]], rows_v, sem).start()

    def gather_wait(rows_v, sem):
        pltpu.make_async_copy(
            table_hbm.at[idx_v.at[pl.ds(0, CH)]], rows_v, sem).wait()

    def out_start(c, rows_v, sem):
        pltpu.make_async_copy(
            rows_v, out_hbm.at[pl.ds(slab + c * CH, CH)], sem).start()

    def out_wait(rows_v, sem):
        pltpu.make_async_copy(
            rows_v, out_hbm.at[pl.ds(slab, CH)], sem).wait()

    def compute(rows_v):
        zero = jnp.zeros((L,), jnp.float32)

        def pass1(j, carry):
            col = jnp.full((L,), j, jnp.int32)
            out = []
            for g in range(NG):
                s, sq = carry[2 * g], carry[2 * g + 1]
                v = plsc.load_gather(rows_v, [row_ids[g], col])
                out.append(s + v)
                out.append(sq + v * v)
            return tuple(out)

        acc = lax.fori_loop(0, D, pass1, (zero,) * (2 * NG))

        stats = []
        for g in range(NG):
            mean = acc[2 * g] * (1.0 / D)
            var = acc[2 * g + 1] * (1.0 / D) - mean * mean
            stats.append((mean, _rsqrt(var + EPS)))

        def pass2(j, carry):
            col = jnp.full((L,), j, jnp.int32)
            gj = g_v[j, :]
            bj = b_v[j, :]
            for g in range(NG):
                mean, rstd = stats[g]
                v = plsc.load_gather(rows_v, [row_ids[g], col])
                o = (v - mean) * rstd * gj + bj
                plsc.store_scatter(rows_v, [row_ids[g], col], o)
            return carry

        lax.fori_loop(0, D, pass2, 0)

    gather(0, rows0, gsem0)
    gather(1, rows1, gsem1)

    def pair_body(p, carry):
        c0 = 2 * p
        c1 = c0 + 1
        gather_wait(rows0, gsem0)
        compute(rows0)
        out_start(c0, rows0, osem0)
        gather_wait(rows1, gsem1)
        compute(rows1)
        out_start(c1, rows1, osem1)

        @pl.when(c0 + 2 < n_chunks)
        def _():
            out_wait(rows0, osem0)
            gather(c0 + 2, rows0, gsem0)

        @pl.when(c1 + 2 < n_chunks)
        def _():
            out_wait(rows1, osem1)
            gather(c1 + 2, rows1, gsem1)

        return carry

    lax.fori_loop(0, n_chunks // 2, pair_body, 0)
    out_wait(rows0, osem0)
    out_wait(rows1, osem1)


def kernel(ticker_ids, table, gamma, beta):
    if ticker_ids.ndim == 2 and ticker_ids.shape[-1] == 1:
        ticker_ids = ticker_ids[..., 0]
    out_shape = ticker_ids.shape + (D,)
    flat = ticker_ids.reshape(-1).astype(jnp.int32)
    n = flat.shape[0]

    unit = NW * CH * 2
    n_pad = ((n + unit - 1) // unit) * unit
    if n_pad != n:
        flat = jnp.concatenate(
            [flat, jnp.zeros((n_pad - n,), jnp.int32)])
    per_w = n_pad // NW

    g_t = jnp.broadcast_to(gamma.astype(jnp.float32)[:, None], (D, L))
    b_t = jnp.broadcast_to(beta.astype(jnp.float32)[:, None], (D, L))

    mesh = plsc.VectorSubcoreMesh(core_axis_name="c", subcore_axis_name="s",
                                  num_cores=2, num_subcores=16)
    run = pl.kernel(
        functools.partial(_sc_body, per_w),
        out_type=jax.ShapeDtypeStruct((n_pad, D), jnp.float32),
        mesh=mesh,
        compiler_params=pltpu.CompilerParams(needs_layout_passes=False,
                                             use_tc_tiling_on_sc=False),
        scratch_types=[
            pltpu.VMEM((per_w,), jnp.int32),
            pltpu.VMEM((CH, D), jnp.float32),
            pltpu.VMEM((CH, D), jnp.float32),
            pltpu.VMEM((D, L), jnp.float32),
            pltpu.VMEM((D, L), jnp.float32),
            pltpu.SemaphoreType.DMA,
            pltpu.SemaphoreType.DMA,
            pltpu.SemaphoreType.DMA,
            pltpu.SemaphoreType.DMA,
        ],
    )
    out = run(table.astype(jnp.float32), flat, g_t, b_t)
    return out[:n].reshape(out_shape)


# speedup vs baseline: 1.2856x; 1.2842x over previous
import functools

import jax
import jax.numpy as jnp
from jax import lax
from jax.experimental import pallas as pl
from jax.experimental.pallas import tpu as pltpu
from jax.experimental.pallas import tpu_sc as plsc

D = 32
L = 16
NW = 32
CH = 128
NG = CH // L
EPS = 1e-5


def _rsqrt(x):
    i = plsc.bitcast(x, jnp.int32)
    i = jnp.int32(0x5F3759DF) - lax.shift_right_logical(i, 1)
    y = plsc.bitcast(i, jnp.float32)
    for _ in range(3):
        y = y * (1.5 - 0.5 * x * y * y)
    return y


NBUF = 4


def _sc_body(per_w, table_hbm, idx_hbm, g_hbm, b_hbm, out_hbm,
             idx_v, rows, g_v, b_v, gsems, osems):
    wid = lax.axis_index("s") * 2 + lax.axis_index("c")
    slab = wid * per_w

    pltpu.sync_copy(idx_hbm.at[pl.ds(slab, per_w)], idx_v)
    pltpu.sync_copy(g_hbm, g_v)
    pltpu.sync_copy(b_hbm, b_v)

    n_chunks = per_w // CH
    lane_iota = lax.iota(jnp.int32, L)
    row_ids = [lane_iota + g * L for g in range(NG)]

    def gather(c, k):
        pltpu.make_async_copy(
            table_hbm.at[idx_v.at[pl.ds(c * CH, CH)]], rows[k], gsems[k]
        ).start()

    def gather_wait(k):
        pltpu.make_async_copy(
            table_hbm.at[idx_v.at[pl.ds(0, CH)]], rows[k], gsems[k]).wait()

    def out_start(c, k):
        pltpu.make_async_copy(
            rows[k], out_hbm.at[pl.ds(slab + c * CH, CH)], osems[k]).start()

    def out_wait(k):
        pltpu.make_async_copy(
            rows[k], out_hbm.at[pl.ds(slab, CH)], osems[k]).wait()

    def compute(rows_v):
        zero = jnp.zeros((L,), jnp.float32)

        @plsc.parallel_loop(0, D, unroll=4, carry=(zero,) * (2 * NG))
        def acc(j, carry):
            col = jnp.full((L,), j, jnp.int32)
            out = []
            for g in range(NG):
                v = plsc.load_gather(rows_v, [row_ids[g], col])
                out.append(carry[2 * g] + v)
                out.append(carry[2 * g + 1] + v * v)
            return tuple(out)

        stats = []
        for g in range(NG):
            mean = acc[2 * g] * (1.0 / D)
            var = acc[2 * g + 1] * (1.0 / D) - mean * mean
            stats.append((mean, _rsqrt(var + EPS)))

        @plsc.parallel_loop(0, D, unroll=4)
        def _(j):
            col = jnp.full((L,), j, jnp.int32)
            gj = g_v[j, :]
            bj = b_v[j, :]
            for g in range(NG):
                mean, rstd = stats[g]
                v = plsc.load_gather(rows_v, [row_ids[g], col])
                o = (v - mean) * rstd * gj + bj
                plsc.store_scatter(rows_v, [row_ids[g], col], o)

    for k in range(NBUF):
        gather(k, k)

    def ring_body(p, carry):
        base = NBUF * p
        for k in range(NBUF):
            gather_wait(k)
            compute(rows[k])
            out_start(base + k, k)
            kp = k - 1 if k else NBUF - 1
            cp = base + kp if k else base - 1

            @pl.when((cp >= 0) & (cp + NBUF < n_chunks))
            def _():
                out_wait(kp)
                gather(cp + NBUF, kp)

        return carry

    lax.fori_loop(0, n_chunks // NBUF, ring_body, 0)
    for k in range(NBUF):
        out_wait(k)


def kernel(ticker_ids, table, gamma, beta):
    if ticker_ids.ndim == 2 and ticker_ids.shape[-1] == 1:
        ticker_ids = ticker_ids[..., 0]
    out_shape = ticker_ids.shape + (D,)
    flat = ticker_ids.reshape(-1).astype(jnp.int32)
    n = flat.shape[0]

    unit = NW * CH * NBUF
    n_pad = ((n + unit - 1) // unit) * unit
    if n_pad != n:
        flat = jnp.concatenate(
            [flat, jnp.zeros((n_pad - n,), jnp.int32)])
    per_w = n_pad // NW

    g_t = jnp.broadcast_to(gamma.astype(jnp.float32)[:, None], (D, L))
    b_t = jnp.broadcast_to(beta.astype(jnp.float32)[:, None], (D, L))

    mesh = plsc.VectorSubcoreMesh(core_axis_name="c", subcore_axis_name="s",
                                  num_cores=2, num_subcores=16)
    run = pl.kernel(
        functools.partial(_sc_body, per_w),
        out_type=jax.ShapeDtypeStruct((n_pad, D), jnp.float32),
        mesh=mesh,
        compiler_params=pltpu.CompilerParams(needs_layout_passes=False,
                                             use_tc_tiling_on_sc=False),
        scratch_types=[
            pltpu.VMEM((per_w,), jnp.int32),
            [pltpu.VMEM((CH, D), jnp.float32) for _ in range(NBUF)],
            pltpu.VMEM((D, L), jnp.float32),
            pltpu.VMEM((D, L), jnp.float32),
            [pltpu.SemaphoreType.DMA for _ in range(NBUF)],
            [pltpu.SemaphoreType.DMA for _ in range(NBUF)],
        ],
    )
    out = run(table.astype(jnp.float32), flat, g_t, b_t)
    return out[:n].reshape(out_shape)


# speedup vs baseline: 2.0798x; 1.6177x over previous
import functools

import jax
import jax.numpy as jnp
from jax import lax
from jax.experimental import pallas as pl
from jax.experimental.pallas import tpu as pltpu
from jax.experimental.pallas import tpu_sc as plsc

D = 32
L = 16
NW = 32
EPS = 1e-5
B_BLK = 16
SC_R = 80
NG = SC_R // L


def _rsqrt(x):
    i = plsc.bitcast(x, jnp.int32)
    i = jnp.int32(0x5F3759DF) - lax.shift_right_logical(i, 1)
    y = plsc.bitcast(i, jnp.float32)
    for _ in range(3):
        y = y * (1.5 - 0.5 * x * y * y)
    return y


def _stats(acc):
    out = []
    for g in range(len(acc) // 2):
        mean = acc[2 * g] * (1.0 / D)
        var = acc[2 * g + 1] * (1.0 / D) - mean * mean
        out.append((mean, _rsqrt(var + EPS)))
    return out


def _sc_body(batch, hist, table_hbm, idx_hbm, g_hbm, b_hbm, out_hbm,
             idx_v, gbufs, tbufs, g_v, b_v, gsems, osems):
    bpw = batch // NW
    per_w = bpw * hist
    n_blk = bpw // B_BLK
    blk_rows = B_BLK * hist
    n_sub = blk_rows // SC_R
    n_sub_tot = per_w // SC_R
    assert n_sub % 2 == 0

    wid = lax.axis_index("s") * 2 + lax.axis_index("c")
    slab = wid * per_w

    pltpu.sync_copy(idx_hbm.at[pl.ds(slab, per_w)], idx_v)
    pltpu.sync_copy(g_hbm, g_v)
    pltpu.sync_copy(b_hbm, b_v)

    lane_iota = lax.iota(jnp.int32, L)

    def gather(s, q):
        pltpu.make_async_copy(
            table_hbm.at[idx_v.at[pl.ds(s * SC_R, SC_R)]], gbufs[q], gsems[q]
        ).start()

    def gather_wait(q):
        pltpu.make_async_copy(
            table_hbm.at[idx_v.at[pl.ds(0, SC_R)]], gbufs[q], gsems[q]).wait()

    def out_start(blk, k):
        pltpu.make_async_copy(
            tbufs[k],
            out_hbm.at[:, :, pl.ds(wid * bpw + blk * B_BLK, B_BLK)],
            osems[k]).start()

    def out_wait(k):
        pltpu.make_async_copy(
            tbufs[k],
            out_hbm.at[:, :, pl.ds(0, B_BLK)], osems[k]).wait()

    def compute(sub, rows_v, t_buf):
        l_ids = [sub * SC_R + g * L + lane_iota for g in range(NG)]
        h_ids = [l % hist for l in l_ids]
        b_ids = [l // hist for l in l_ids]
        row_ids = [lane_iota + g * L for g in range(NG)]
        zero = jnp.zeros((L,), jnp.float32)

        @plsc.parallel_loop(0, D, unroll=2, carry=(zero,) * (2 * NG))
        def acc(j, carry):
            col = jnp.full((L,), j, jnp.int32)
            out = []
            for g in range(NG):
                v = plsc.load_gather(rows_v, [row_ids[g], col])
                out.append(carry[2 * g] + v)
                out.append(carry[2 * g + 1] + v * v)
            return tuple(out)

        stats = _stats(acc)

        @plsc.parallel_loop(0, D, unroll=2)
        def _(j):
            col = jnp.full((L,), j, jnp.int32)
            gj = g_v[j, :]
            bj = b_v[j, :]
            for g in range(NG):
                mean, rstd = stats[g]
                v = plsc.load_gather(rows_v, [row_ids[g], col])
                o = (v - mean) * rstd * gj + bj
                plsc.store_scatter(t_buf, [h_ids[g], col, b_ids[g]], o)

    gather(0, 0)
    gather(1, 1)

    def pair_body(p, carry):
        for kb in range(2):
            blk = 2 * p + kb

            @pl.when(p > 0)
            def _():
                out_wait(kb)

            for s2 in range(n_sub // 2):
                for k2 in range(2):
                    sub = 2 * s2 + k2
                    s = blk * n_sub + sub
                    gather_wait(k2)
                    compute(sub, gbufs[k2], tbufs[kb])

                    @pl.when(s + 2 < n_sub_tot)
                    def _():
                        gather(s + 2, k2)

            out_start(blk, kb)
        return carry

    lax.fori_loop(0, n_blk // 2, pair_body, 0)
    out_wait(0)
    out_wait(1)


def kernel(ticker_ids, table, gamma, beta):
    if ticker_ids.ndim == 2 and ticker_ids.shape[-1] == 1:
        ticker_ids = ticker_ids[..., 0]
    assert ticker_ids.ndim == 2
    batch, hist = ticker_ids.shape
    assert batch % (NW * B_BLK * 2) == 0 and (B_BLK * hist) % (2 * SC_R) == 0

    flat = ticker_ids.reshape(-1).astype(jnp.int32)

    g_t = jnp.broadcast_to(gamma.astype(jnp.float32)[:, None], (D, L))
    b_t = jnp.broadcast_to(beta.astype(jnp.float32)[:, None], (D, L))

    per_w = (batch // NW) * hist
    mesh = plsc.VectorSubcoreMesh(core_axis_name="c", subcore_axis_name="s",
                                  num_cores=2, num_subcores=16)
    run = pl.kernel(
        functools.partial(_sc_body, batch, hist),
        out_type=jax.ShapeDtypeStruct((hist, D, batch), jnp.float32),
        mesh=mesh,
        compiler_params=pltpu.CompilerParams(needs_layout_passes=False,
                                             use_tc_tiling_on_sc=False),
        scratch_types=[
            pltpu.VMEM((per_w,), jnp.int32),
            [pltpu.VMEM((SC_R, D), jnp.float32) for _ in range(2)],
            [pltpu.VMEM((hist, D, B_BLK), jnp.float32) for _ in range(2)],
            pltpu.VMEM((D, L), jnp.float32),
            pltpu.VMEM((D, L), jnp.float32),
            [pltpu.SemaphoreType.DMA for _ in range(2)],
            [pltpu.SemaphoreType.DMA for _ in range(2)],
        ],
    )
    out = run(table.astype(jnp.float32), flat, g_t, b_t)
    return out.transpose(2, 0, 1)
